# baseline (speedup 1.0000x reference)
import functools

import jax
import jax.numpy as jnp
from jax import lax
from jax.experimental import pallas as pl
from jax.experimental.pallas import tpu as pltpu
from jax.experimental.pallas import tpu_sc as plsc

N = 10000
E = 320000
D = 128
NP = 10240
NC = 2
NS = 16
NW = NC * NS
EP = 327680
K = 128
RD = 80
SEGW = 16
ASH = NP + 16
AZT = ASH // NS

_mesh = plsc.VectorSubcoreMesh(core_axis_name="c", subcore_axis_name="s")
_HIGH = lax.Precision.HIGHEST
_SC_PARAMS = pltpu.CompilerParams(needs_layout_passes=False)


def _zero_1d(ref, n):
    z = jnp.zeros((16,), ref.dtype)

    def body(i, _):
        ref[pl.ds(i * 16, 16)] = z
        return 0

    lax.fori_loop(0, n // 16, body, 0)


@functools.partial(
    pl.kernel,
    out_type=jax.ShapeDtypeStruct((2, NW, NP), jnp.float32),
    mesh=_mesh,
    compiler_params=_SC_PARAMS,
    scratch_types=[
        pltpu.VMEM((RD, K), jnp.int32),
        pltpu.VMEM((RD, K), jnp.int32),
        pltpu.VMEM((NP,), jnp.float32),
        pltpu.VMEM((NP,), jnp.float32),
    ],
)
def _sc_degrees(edges, out, srcv, dstv, dego, degi):
    cid = lax.axis_index("c")
    sid = lax.axis_index("s")
    wid = cid * NS + sid
    pltpu.sync_copy(edges.at[0, wid], srcv)
    pltpu.sync_copy(edges.at[1, wid], dstv)
    _zero_1d(dego, NP)
    _zero_1d(degi, NP)
    ones = jnp.ones((16,), jnp.float32)
    lanes = lax.iota(jnp.int32, 16)
    tile_base = wid * (RD * K)

    def body(m, _):
        for q in range(K // 16):
            pos = tile_base + m * K + q * 16 + lanes
            live = pos < E
            s16 = srcv[m, pl.ds(q * 16, 16)]
            d16 = dstv[m, pl.ds(q * 16, 16)]
            plsc.addupdate_scatter(dego, [s16], ones, mask=live)
            plsc.addupdate_scatter(degi, [d16], ones, mask=live)
        return 0

    lax.fori_loop(0, RD, body, 0)
    pltpu.sync_copy(dego, out.at[0, wid])
    pltpu.sync_copy(degi, out.at[1, wid])


def _isqrt_body(dp_ref, isq_ref):
    s = jnp.sum(dp_ref[...], axis=1)
    isq_ref[...] = lax.rsqrt(jnp.maximum(s, 1.0))


def _tc_isqrt(deg_p):
    return pl.pallas_call(
        _isqrt_body,
        out_shape=jax.ShapeDtypeStruct((2, NP), jnp.float32),
    )(deg_p)


def _scale_body(f_ref, oi_ref, out_ref):
    out_ref[...] = f_ref[...] * oi_ref[...]


def _tc_scale(features, oi_col):
    br = 2000
    return pl.pallas_call(
        _scale_body,
        grid=(N // br,),
        in_specs=[
            pl.BlockSpec((br, D), lambda i: (i, 0)),
            pl.BlockSpec((br, 1), lambda i: (i, 0)),
        ],
        out_specs=pl.BlockSpec((br, D), lambda i: (i, 0)),
        out_shape=jax.ShapeDtypeStruct((N, D), jnp.float32),
    )(features, oi_col)


@functools.partial(
    pl.kernel,
    out_type=jax.ShapeDtypeStruct((NW, NP), jnp.float32),
    mesh=_mesh,
    compiler_params=_SC_PARAMS,
    scratch_types=[
        pltpu.VMEM((RD, K), jnp.int32),
        pltpu.VMEM((RD, K), jnp.int32),
        pltpu.VMEM((NP + 16,), jnp.float32),
        pltpu.VMEM((NP,), jnp.float32),
    ],
)
def _sc_cvec(edges, isq, c_out, srcv, dstv, iiv, cpart):
    cid = lax.axis_index("c")
    sid = lax.axis_index("s")
    wid = cid * NS + sid
    pltpu.sync_copy(edges.at[0, wid], srcv)
    pltpu.sync_copy(edges.at[1, wid], dstv)
    pltpu.sync_copy(isq.at[1], iiv.at[pl.ds(0, NP)])
    iiv[pl.ds(NP, 16)] = jnp.zeros((16,), jnp.float32)
    _zero_1d(cpart, NP)

    def body(m, _):
        for q in range(K // 16):
            s16 = srcv[m, pl.ds(q * 16, 16)]
            d16 = dstv[m, pl.ds(q * 16, 16)]
            wv = plsc.load_gather(iiv, [d16])
            plsc.addupdate_scatter(cpart, [s16], wv)
        return 0

    lax.fori_loop(0, RD, body, 0)
    pltpu.sync_copy(cpart, c_out.at[wid])


@functools.partial(
    pl.kernel,
    out_type=jax.ShapeDtypeStruct((NC, NP, D), jnp.float32),
    mesh=_mesh,
    compiler_params=_SC_PARAMS,
    scratch_types=[
        pltpu.VMEM((SEGW, K), jnp.int32),
        pltpu.VMEM((SEGW, K), jnp.int32),
        pltpu.VMEM((K, D), jnp.float32),
        pltpu.VMEM((K, D), jnp.float32),
        pltpu.VMEM_SHARED((ASH, D), jnp.float32),
        pltpu.SemaphoreType.DMA,
        pltpu.SemaphoreType.DMA,
    ],
)
def _sc_aggregate(edges, h0n, agg_out, sidx, didx, rows0, rows1, agg_sh,
                  gsem, ssem):
    cid = lax.axis_index("c")
    sid = lax.axis_index("s")
    wid = cid * NS + sid
    z16 = jnp.zeros((16,), jnp.float32)

    def zrow(r, _):
        for q in range(D // 16):
            rows0[r, pl.ds(q * 16, 16)] = z16
        return 0

    lax.fori_loop(0, K, zrow, 0)
    for blk in range(AZT // K):
        pltpu.sync_copy(rows0, agg_sh.at[pl.ds(sid * AZT + blk * K, K)])
    pltpu.sync_copy(rows0.at[pl.ds(0, AZT - (AZT // K) * K)],
                    agg_sh.at[pl.ds(sid * AZT + (AZT // K) * K,
                                    AZT - (AZT // K) * K)])
    plsc.subcore_barrier()

    rows = [rows0, rows1]

    def seg(sg, _):
        pltpu.sync_copy(edges.at[0, wid, pl.ds(sg * SEGW, SEGW)], sidx)
        pltpu.sync_copy(edges.at[1, wid, pl.ds(sg * SEGW, SEGW)], didx)

        def wave(w, _):
            gh = [pltpu.async_copy(h0n.at[sidx.at[w * 2 + b]], rows[b],
                                   gsem)
                  for b in range(2)]
            sh = []
            for b in range(2):
                gh[b].wait()
                sh.append(pltpu.async_copy(
                    rows[b], agg_sh.at[didx.at[w * 2 + b]], ssem,
                    add=True))
            for b in range(2):
                sh[b].wait()
            return 0

        lax.fori_loop(0, SEGW // 2, wave, 0)
        return 0

    lax.fori_loop(0, RD // SEGW, seg, 0)
    plsc.subcore_barrier()
    pltpu.sync_copy(agg_sh.at[pl.ds(sid * (NP // NS), NP // NS)],
                    agg_out.at[cid, pl.ds(sid * (NP // NS), NP // NS)])


def _finish_body(agg_ref, ii_ref, oi_ref, c_ref, w1_ref, b1_ref, w2_ref,
                 b2_ref, out_ref, s_ref):
    i = pl.program_id(0)

    @pl.when(i == 0)
    def _():
        s_ref[...] = jnp.zeros_like(s_ref)

    aggn = (agg_ref[0] + agg_ref[1]) * ii_ref[...]
    h1 = jnp.maximum(
        jnp.dot(aggn, w1_ref[...], precision=_HIGH) + b1_ref[...], 0.0)
    h1n = h1 * oi_ref[...]
    cs = jnp.sum(c_ref[...], axis=0, keepdims=True)
    s_ref[...] += jnp.dot(cs, h1n, precision=_HIGH)

    @pl.when(i == pl.num_programs(0) - 1)
    def _():
        out_ref[...] = (
            jnp.dot(s_ref[...] * (1.0 / N), w2_ref[...], precision=_HIGH)
            + b2_ref[...])


def _tc_finish(agg, ii_col, oi_col, c_p, W1, b1r, W2, b2r):
    br = 512
    return pl.pallas_call(
        _finish_body,
        grid=(NP // br,),
        in_specs=[
            pl.BlockSpec((NC, br, D), lambda i: (0, i, 0)),
            pl.BlockSpec((br, 1), lambda i: (i, 0)),
            pl.BlockSpec((br, 1), lambda i: (i, 0)),
            pl.BlockSpec((NW, br), lambda i: (0, i)),
            pl.BlockSpec((D, D), lambda i: (0, 0)),
            pl.BlockSpec((1, D), lambda i: (0, 0)),
            pl.BlockSpec((D, D), lambda i: (0, 0)),
            pl.BlockSpec((1, D), lambda i: (0, 0)),
        ],
        out_specs=pl.BlockSpec((1, D), lambda i: (0, 0)),
        out_shape=jax.ShapeDtypeStruct((1, D), jnp.float32),
        scratch_shapes=[pltpu.VMEM((1, D), jnp.float32)],
    )(agg, ii_col, oi_col, c_p, W1, b1r, W2, b2r)


@jax.jit
def kernel(features, edge_index, W1, b1, W2, b2):
    ei = edge_index.astype(jnp.int32)
    ep = jnp.concatenate(
        [ei, jnp.stack([jnp.zeros((EP - E,), jnp.int32),
                        jnp.full((EP - E,), NP, jnp.int32)])], axis=1)
    ep = ep.reshape(2, NW, RD, K)
    deg_p = _sc_degrees(ep)
    isq = _tc_isqrt(deg_p)
    oi_n = isq[0, :N].reshape(N, 1)
    h0n = _tc_scale(features, oi_n)
    c_p = _sc_cvec(ep, isq)
    agg = _sc_aggregate(ep, h0n)
    ii_col = isq[1].reshape(NP, 1)
    oi_col = isq[0].reshape(NP, 1)
    return _tc_finish(agg, ii_col, oi_col, c_p, W1,
                      b1.reshape(1, D), W2, b2.reshape(1, D))


# speedup vs baseline: 2.3381x; 2.3381x over previous
import functools

import jax
import jax.numpy as jnp
from jax import lax
from jax.experimental import pallas as pl
from jax.experimental.pallas import tpu as pltpu
from jax.experimental.pallas import tpu_sc as plsc

N = 10000
E = 320000
D = 128
NP = 10240
NC = 2
NS = 16
NW = NC * NS
EP = 327680
K = 128
RD = 80
SEGW = 16
ASH = NP + 16
AZT = ASH // NS

_mesh = plsc.VectorSubcoreMesh(core_axis_name="c", subcore_axis_name="s")
_HIGH = lax.Precision.HIGHEST
_SC_PARAMS = pltpu.CompilerParams(needs_layout_passes=False)


def _zero_1d(ref, n):
    z = jnp.zeros((16,), ref.dtype)

    def body(i, _):
        ref[pl.ds(i * 16, 16)] = z
        return 0

    lax.fori_loop(0, n // 16, body, 0)


@functools.partial(
    pl.kernel,
    out_type=jax.ShapeDtypeStruct((2, NW, NP), jnp.float32),
    mesh=_mesh,
    compiler_params=_SC_PARAMS,
    scratch_types=[
        pltpu.VMEM((RD, K), jnp.int32),
        pltpu.VMEM((RD, K), jnp.int32),
        pltpu.VMEM((NP,), jnp.float32),
        pltpu.VMEM((NP,), jnp.float32),
    ],
)
def _sc_degrees(edges, out, srcv, dstv, dego, degi):
    cid = lax.axis_index("c")
    sid = lax.axis_index("s")
    wid = cid * NS + sid
    pltpu.sync_copy(edges.at[0, wid], srcv)
    pltpu.sync_copy(edges.at[1, wid], dstv)
    _zero_1d(dego, NP)
    _zero_1d(degi, NP)
    ones = jnp.ones((16,), jnp.float32)
    lanes = lax.iota(jnp.int32, 16)
    tile_base = wid * (RD * K)

    def body(m, _):
        for q in range(K // 16):
            pos = tile_base + m * K + q * 16 + lanes
            live = pos < E
            s16 = srcv[m, pl.ds(q * 16, 16)]
            d16 = dstv[m, pl.ds(q * 16, 16)]
            plsc.addupdate_scatter(dego, [s16], ones, mask=live)
            plsc.addupdate_scatter(degi, [d16], ones, mask=live)
        return 0

    lax.fori_loop(0, RD, body, 0)
    pltpu.sync_copy(dego, out.at[0, wid])
    pltpu.sync_copy(degi, out.at[1, wid])


def _isqrt_body(dp_ref, isq_ref):
    s = jnp.sum(dp_ref[...], axis=1)
    isq_ref[...] = lax.rsqrt(jnp.maximum(s, 1.0))


def _tc_isqrt(deg_p):
    return pl.pallas_call(
        _isqrt_body,
        out_shape=jax.ShapeDtypeStruct((2, NP), jnp.float32),
    )(deg_p)


def _scale_body(f_ref, oi_ref, out_ref):
    out_ref[...] = f_ref[...] * oi_ref[...]


def _tc_scale(features, oi_col):
    br = 2000
    return pl.pallas_call(
        _scale_body,
        grid=(N // br,),
        in_specs=[
            pl.BlockSpec((br, D), lambda i: (i, 0)),
            pl.BlockSpec((br, 1), lambda i: (i, 0)),
        ],
        out_specs=pl.BlockSpec((br, D), lambda i: (i, 0)),
        out_shape=jax.ShapeDtypeStruct((N, D), jnp.float32),
    )(features, oi_col)


@functools.partial(
    pl.kernel,
    out_type=jax.ShapeDtypeStruct((NW, NP), jnp.float32),
    mesh=_mesh,
    compiler_params=_SC_PARAMS,
    scratch_types=[
        pltpu.VMEM((RD, K), jnp.int32),
        pltpu.VMEM((RD, K), jnp.int32),
        pltpu.VMEM((NP + 16,), jnp.float32),
        pltpu.VMEM((NP,), jnp.float32),
    ],
)
def _sc_cvec(edges, isq, c_out, srcv, dstv, iiv, cpart):
    cid = lax.axis_index("c")
    sid = lax.axis_index("s")
    wid = cid * NS + sid
    pltpu.sync_copy(edges.at[0, wid], srcv)
    pltpu.sync_copy(edges.at[1, wid], dstv)
    pltpu.sync_copy(isq.at[1], iiv.at[pl.ds(0, NP)])
    iiv[pl.ds(NP, 16)] = jnp.zeros((16,), jnp.float32)
    _zero_1d(cpart, NP)

    lc = jnp.clip(E // K - wid * RD, 0, RD)

    def body(m, _):
        for q in range(K // 16):
            s16 = srcv[m, pl.ds(q * 16, 16)]
            d16 = dstv[m, pl.ds(q * 16, 16)]
            wv = plsc.load_gather(iiv, [d16])
            plsc.addupdate_scatter(cpart, [s16], wv)
        return 0

    lax.fori_loop(0, lc, body, 0)
    pltpu.sync_copy(cpart, c_out.at[wid])


@functools.partial(
    pl.kernel,
    out_type=jax.ShapeDtypeStruct((NC, NP, D), jnp.float32),
    mesh=_mesh,
    compiler_params=_SC_PARAMS,
    scratch_types=[
        pltpu.VMEM((SEGW, K), jnp.int32),
        pltpu.VMEM((SEGW, K), jnp.int32),
        pltpu.VMEM((K, D), jnp.float32),
        pltpu.VMEM((K, D), jnp.float32),
        pltpu.VMEM_SHARED((ASH, D), jnp.float32),
        pltpu.SemaphoreType.DMA,
        pltpu.SemaphoreType.DMA,
    ],
)
def _sc_aggregate(edges, h0n, agg_out, sidx, didx, rows0, rows1, agg_sh,
                  gsem, ssem):
    cid = lax.axis_index("c")
    sid = lax.axis_index("s")
    wid = cid * NS + sid
    z16 = jnp.zeros((16,), jnp.float32)

    def zrow(r, _):
        for q in range(D // 16):
            rows0[r, pl.ds(q * 16, 16)] = z16
        return 0

    lax.fori_loop(0, K, zrow, 0)
    for blk in range(AZT // K):
        pltpu.sync_copy(rows0, agg_sh.at[pl.ds(sid * AZT + blk * K, K)])
    pltpu.sync_copy(rows0.at[pl.ds(0, AZT - (AZT // K) * K)],
                    agg_sh.at[pl.ds(sid * AZT + (AZT // K) * K,
                                    AZT - (AZT // K) * K)])
    plsc.subcore_barrier()

    rows = [rows0, rows1]
    lc = jnp.clip(E // K - wid * RD, 0, RD)

    def seg(sg, _):
        pltpu.sync_copy(edges.at[0, wid, pl.ds(sg * SEGW, SEGW)], sidx)
        pltpu.sync_copy(edges.at[1, wid, pl.ds(sg * SEGW, SEGW)], didx)

        def wave(w, _):
            gh = [pltpu.async_copy(h0n.at[sidx.at[w * 2 + b]], rows[b],
                                   gsem)
                  for b in range(2)]
            sh = []
            for b in range(2):
                gh[b].wait()
                sh.append(pltpu.async_copy(
                    rows[b], agg_sh.at[didx.at[w * 2 + b]], ssem,
                    add=True))
            for b in range(2):
                sh[b].wait()
            return 0

        nw = jnp.clip(lc - sg * SEGW, 0, SEGW) // 2
        lax.fori_loop(0, nw, wave, 0)
        return 0

    lax.fori_loop(0, RD // SEGW, seg, 0)
    plsc.subcore_barrier()
    pltpu.sync_copy(agg_sh.at[pl.ds(sid * (NP // NS), NP // NS)],
                    agg_out.at[cid, pl.ds(sid * (NP // NS), NP // NS)])


def _finish_body(agg_ref, ii_ref, oi_ref, c_ref, w1_ref, b1_ref, w2_ref,
                 b2_ref, out_ref, s_ref):
    i = pl.program_id(0)

    @pl.when(i == 0)
    def _():
        s_ref[...] = jnp.zeros_like(s_ref)

    aggn = (agg_ref[0] + agg_ref[1]) * ii_ref[...]
    h1 = jnp.maximum(
        jnp.dot(aggn, w1_ref[...], precision=_HIGH) + b1_ref[...], 0.0)
    h1n = h1 * oi_ref[...]
    cs = jnp.sum(c_ref[...], axis=0, keepdims=True)
    s_ref[...] += jnp.dot(cs, h1n, precision=_HIGH)

    @pl.when(i == pl.num_programs(0) - 1)
    def _():
        out_ref[...] = (
            jnp.dot(s_ref[...] * (1.0 / N), w2_ref[...], precision=_HIGH)
            + b2_ref[...])


def _tc_finish(agg, ii_col, oi_col, c_p, W1, b1r, W2, b2r):
    br = 512
    return pl.pallas_call(
        _finish_body,
        grid=(NP // br,),
        in_specs=[
            pl.BlockSpec((NC, br, D), lambda i: (0, i, 0)),
            pl.BlockSpec((br, 1), lambda i: (i, 0)),
            pl.BlockSpec((br, 1), lambda i: (i, 0)),
            pl.BlockSpec((NW, br), lambda i: (0, i)),
            pl.BlockSpec((D, D), lambda i: (0, 0)),
            pl.BlockSpec((1, D), lambda i: (0, 0)),
            pl.BlockSpec((D, D), lambda i: (0, 0)),
            pl.BlockSpec((1, D), lambda i: (0, 0)),
        ],
        out_specs=pl.BlockSpec((1, D), lambda i: (0, 0)),
        out_shape=jax.ShapeDtypeStruct((1, D), jnp.float32),
        scratch_shapes=[pltpu.VMEM((1, D), jnp.float32)],
    )(agg, ii_col, oi_col, c_p, W1, b1r, W2, b2r)


@jax.jit
def kernel(features, edge_index, W1, b1, W2, b2):
    ei = edge_index.astype(jnp.int32)
    ep = jnp.concatenate(
        [ei, jnp.stack([jnp.zeros((EP - E,), jnp.int32),
                        jnp.full((EP - E,), NP, jnp.int32)])], axis=1)
    ep = ep.reshape(2, NW, RD, K)
    deg_p = _sc_degrees(ep)
    isq = _tc_isqrt(deg_p)
    oi_n = isq[0, :N].reshape(N, 1)
    h0n = _tc_scale(features, oi_n)
    c_p = _sc_cvec(ep, isq)
    agg = _sc_aggregate(ep, h0n)
    ii_col = isq[1].reshape(NP, 1)
    oi_col = isq[0].reshape(NP, 1)
    return _tc_finish(agg, ii_col, oi_col, c_p, W1,
                      b1.reshape(1, D), W2, b2.reshape(1, D))


# speedup vs baseline: 2.6075x; 1.1152x over previous
import functools

import jax
import jax.numpy as jnp
from jax import lax
from jax.experimental import pallas as pl
from jax.experimental.pallas import tpu as pltpu
from jax.experimental.pallas import tpu_sc as plsc

N = 10000
E = 320000
D = 128
NP = 10240
NC = 2
NS = 16
NW = NC * NS
EP = 327680
K = 128
RD = 80
LCK = E // K
SEGW = 16
ASH = NP + 16
AZT = ASH // NS

_mesh = plsc.VectorSubcoreMesh(core_axis_name="c", subcore_axis_name="s")
_HIGH = lax.Precision.HIGHEST
_SC_PARAMS = pltpu.CompilerParams(needs_layout_passes=False)


def _zero_1d(ref, n):
    z = jnp.zeros((16,), ref.dtype)

    def body(i, _):
        ref[pl.ds(i * 16, 16)] = z
        return 0

    lax.fori_loop(0, n // 16, body, 0)


@functools.partial(
    pl.kernel,
    out_type=jax.ShapeDtypeStruct((2, NW, NP), jnp.float32),
    mesh=_mesh,
    compiler_params=_SC_PARAMS,
    scratch_types=[
        pltpu.VMEM((RD, K), jnp.int32),
        pltpu.VMEM((RD, K), jnp.int32),
        pltpu.VMEM((NP,), jnp.float32),
        pltpu.VMEM((NP,), jnp.float32),
    ],
)
def _sc_degrees(edges, out, srcv, dstv, dego, degi):
    cid = lax.axis_index("c")
    sid = lax.axis_index("s")
    wid = cid * NS + sid
    pltpu.sync_copy(edges.at[0, wid], srcv)
    pltpu.sync_copy(edges.at[1, wid], dstv)
    _zero_1d(dego, NP)
    _zero_1d(degi, NP)
    ones = jnp.ones((16,), jnp.float32)
    lanes = lax.iota(jnp.int32, 16)
    tile_base = wid * (RD * K)

    def body(m, _):
        for q in range(K // 16):
            pos = tile_base + m * K + q * 16 + lanes
            live = pos < E
            s16 = srcv[m, pl.ds(q * 16, 16)]
            d16 = dstv[m, pl.ds(q * 16, 16)]
            plsc.addupdate_scatter(dego, [s16], ones, mask=live)
            plsc.addupdate_scatter(degi, [d16], ones, mask=live)
        return 0

    lax.fori_loop(0, RD, body, 0)
    pltpu.sync_copy(dego, out.at[0, wid])
    pltpu.sync_copy(degi, out.at[1, wid])


def _isqrt_body(dp_ref, isq_ref):
    s = jnp.sum(dp_ref[...], axis=1)
    isq_ref[...] = lax.rsqrt(jnp.maximum(s, 1.0))


def _tc_isqrt(deg_p):
    return pl.pallas_call(
        _isqrt_body,
        out_shape=jax.ShapeDtypeStruct((2, NP), jnp.float32),
    )(deg_p)


def _scale_body(f_ref, oi_ref, out_ref):
    out_ref[...] = f_ref[...] * oi_ref[...]


def _tc_scale(features, oi_col):
    br = 2000
    return pl.pallas_call(
        _scale_body,
        grid=(N // br,),
        in_specs=[
            pl.BlockSpec((br, D), lambda i: (i, 0)),
            pl.BlockSpec((br, 1), lambda i: (i, 0)),
        ],
        out_specs=pl.BlockSpec((br, D), lambda i: (i, 0)),
        out_shape=jax.ShapeDtypeStruct((N, D), jnp.float32),
    )(features, oi_col)


@functools.partial(
    pl.kernel,
    out_type=jax.ShapeDtypeStruct((NW, NP), jnp.float32),
    mesh=_mesh,
    compiler_params=_SC_PARAMS,
    scratch_types=[
        pltpu.VMEM((RD, K), jnp.int32),
        pltpu.VMEM((RD, K), jnp.int32),
        pltpu.VMEM((NP + 16,), jnp.float32),
        pltpu.VMEM((NP,), jnp.float32),
    ],
)
def _sc_cvec(edges, isq, c_out, srcv, dstv, iiv, cpart):
    cid = lax.axis_index("c")
    sid = lax.axis_index("s")
    wid = cid * NS + sid
    pltpu.sync_copy(edges.at[0, wid], srcv)
    pltpu.sync_copy(edges.at[1, wid], dstv)
    pltpu.sync_copy(isq.at[1], iiv.at[pl.ds(0, NP)])
    iiv[pl.ds(NP, 16)] = jnp.zeros((16,), jnp.float32)
    _zero_1d(cpart, NP)

    lc = jnp.clip(LCK - wid * RD, 0, RD)

    def body(m, _):
        for q in range(K // 16):
            s16 = srcv[m, pl.ds(q * 16, 16)]
            d16 = dstv[m, pl.ds(q * 16, 16)]
            wv = plsc.load_gather(iiv, [d16])
            plsc.addupdate_scatter(cpart, [s16], wv)
        return 0

    lax.fori_loop(0, lc, body, 0)
    pltpu.sync_copy(cpart, c_out.at[wid])


@functools.partial(
    pl.kernel,
    out_type=jax.ShapeDtypeStruct((NC, NP, D), jnp.float32),
    mesh=_mesh,
    compiler_params=_SC_PARAMS,
    scratch_types=[
        pltpu.VMEM((2 * SEGW, K), jnp.int32),
        pltpu.VMEM((2 * SEGW, K), jnp.int32),
        pltpu.VMEM((K, D), jnp.float32),
        pltpu.VMEM((K, D), jnp.float32),
        pltpu.VMEM_SHARED((ASH, D), jnp.float32),
        pltpu.SemaphoreType.DMA,
        pltpu.SemaphoreType.DMA,
    ],
)
def _sc_aggregate(edges, h0n, agg_out, sidx, didx, rows0, rows1, agg_sh,
                  gsem, ssem):
    cid = lax.axis_index("c")
    sid = lax.axis_index("s")
    wid = cid * NS + sid
    z16 = jnp.zeros((16,), jnp.float32)

    def zrow(r, _):
        for q in range(D // 16):
            rows0[r, pl.ds(q * 16, 16)] = z16
        return 0

    lax.fori_loop(0, K, zrow, 0)
    for blk in range(AZT // K):
        pltpu.sync_copy(rows0, agg_sh.at[pl.ds(sid * AZT + blk * K, K)])
    pltpu.sync_copy(rows0.at[pl.ds(0, AZT - (AZT // K) * K)],
                    agg_sh.at[pl.ds(sid * AZT + (AZT // K) * K,
                                    AZT - (AZT // K) * K)])
    plsc.subcore_barrier()

    lc = jnp.clip(LCK - wid * RD, 0, RD)
    rows = [rows0, rows1]

    def ldseg(s):
        h = jnp.bitwise_and(s, 1) * SEGW
        pltpu.sync_copy(edges.at[0, wid, pl.ds(s * SEGW, SEGW)],
                        sidx.at[pl.ds(h, SEGW)])
        pltpu.sync_copy(edges.at[1, wid, pl.ds(s * SEGW, SEGW)],
                        didx.at[pl.ds(h, SEGW)])
        return 0

    def gath(c, b):
        return pltpu.async_copy(
            h0n.at[sidx.at[jnp.bitwise_and(c, 2 * SEGW - 1)]], rows[b],
            gsem)

    def scat(c, b):
        return pltpu.async_copy(
            rows[b], agg_sh.at[didx.at[jnp.bitwise_and(c, 2 * SEGW - 1)]],
            ssem, add=True)

    def drain(sem):
        pltpu.make_async_copy(h0n.at[pl.ds(0, K)], rows0, sem).wait()
        return 0

    ldseg(0)

    def wave(w, _):
        c0 = 2 * w
        lax.cond(c0 >= 2, lambda: drain(ssem), lambda: 0)
        lax.cond(jnp.logical_and(jnp.bitwise_and(c0, SEGW - 1) == 0,
                                 c0 > 0),
                 lambda: ldseg(c0 // SEGW), lambda: 0)
        gath(c0, 0)
        lax.cond(c0 >= 1,
                 lambda: (drain(gsem), scat(c0 - 1, 1), 0)[2],
                 lambda: 0)
        lax.cond(c0 >= 1, lambda: drain(ssem), lambda: 0)
        gath(c0 + 1, 1)
        drain(gsem)
        scat(c0, 0)
        return 0

    lax.fori_loop(0, lc // 2, wave, 0)
    drain(gsem)
    scat_last = pltpu.async_copy(
        rows1, agg_sh.at[didx.at[jnp.bitwise_and(lc - 1, 2 * SEGW - 1)]],
        ssem, add=True)
    drain(ssem)
    scat_last.wait()
    plsc.subcore_barrier()
    pltpu.sync_copy(agg_sh.at[pl.ds(sid * (NP // NS), NP // NS)],
                    agg_out.at[cid, pl.ds(sid * (NP // NS), NP // NS)])


def _finish_body(agg_ref, ii_ref, oi_ref, c_ref, w1_ref, b1_ref, w2_ref,
                 b2_ref, out_ref, s_ref):
    i = pl.program_id(0)

    @pl.when(i == 0)
    def _():
        s_ref[...] = jnp.zeros_like(s_ref)

    aggn = (agg_ref[0] + agg_ref[1]) * ii_ref[...]
    h1 = jnp.maximum(
        jnp.dot(aggn, w1_ref[...], precision=_HIGH) + b1_ref[...], 0.0)
    h1n = h1 * oi_ref[...]
    cs = jnp.sum(c_ref[...], axis=0, keepdims=True)
    s_ref[...] += jnp.dot(cs, h1n, precision=_HIGH)

    @pl.when(i == pl.num_programs(0) - 1)
    def _():
        out_ref[...] = (
            jnp.dot(s_ref[...] * (1.0 / N), w2_ref[...], precision=_HIGH)
            + b2_ref[...])


def _tc_finish(agg, ii_col, oi_col, c_p, W1, b1r, W2, b2r):
    br = 512
    return pl.pallas_call(
        _finish_body,
        grid=(NP // br,),
        in_specs=[
            pl.BlockSpec((NC, br, D), lambda i: (0, i, 0)),
            pl.BlockSpec((br, 1), lambda i: (i, 0)),
            pl.BlockSpec((br, 1), lambda i: (i, 0)),
            pl.BlockSpec((NW, br), lambda i: (0, i)),
            pl.BlockSpec((D, D), lambda i: (0, 0)),
            pl.BlockSpec((1, D), lambda i: (0, 0)),
            pl.BlockSpec((D, D), lambda i: (0, 0)),
            pl.BlockSpec((1, D), lambda i: (0, 0)),
        ],
        out_specs=pl.BlockSpec((1, D), lambda i: (0, 0)),
        out_shape=jax.ShapeDtypeStruct((1, D), jnp.float32),
        scratch_shapes=[pltpu.VMEM((1, D), jnp.float32)],
    )(agg, ii_col, oi_col, c_p, W1, b1r, W2, b2r)


@jax.jit
def kernel(features, edge_index, W1, b1, W2, b2):
    ei = edge_index.astype(jnp.int32)
    ep = jnp.concatenate(
        [ei, jnp.stack([jnp.zeros((EP - E,), jnp.int32),
                        jnp.full((EP - E,), NP, jnp.int32)])], axis=1)
    ep = ep.reshape(2, NW, RD, K)
    deg_p = _sc_degrees(ep)
    isq = _tc_isqrt(deg_p)
    oi_n = isq[0, :N].reshape(N, 1)
    h0n = _tc_scale(features, oi_n)
    c_p = _sc_cvec(ep, isq)
    agg = _sc_aggregate(ep, h0n)
    ii_col = isq[1].reshape(NP, 1)
    oi_col = isq[0].reshape(NP, 1)
    return _tc_finish(agg, ii_col, oi_col, c_p, W1,
                      b1.reshape(1, D), W2, b2.reshape(1, D))


# speedup vs baseline: 2.6849x; 1.0297x over previous
import functools

import jax
import jax.numpy as jnp
from jax import lax
from jax.experimental import pallas as pl
from jax.experimental.pallas import tpu as pltpu
from jax.experimental.pallas import tpu_sc as plsc

N = 10000
E = 320000
D = 128
NP = 10240
NC = 2
NS = 16
NW = NC * NS
EP = 327680
K = 128
RD = 80
LCK = E // K
SEGW = 16
ASH = NP + 16
AZT = ASH // NS

_mesh = plsc.VectorSubcoreMesh(core_axis_name="c", subcore_axis_name="s")
_HIGH = lax.Precision.HIGHEST
_SC_PARAMS = pltpu.CompilerParams(needs_layout_passes=False)


def _zero_1d(ref, n):
    z = jnp.zeros((16,), ref.dtype)

    def body(i, _):
        ref[pl.ds(i * 16, 16)] = z
        return 0

    lax.fori_loop(0, n // 16, body, 0)


@functools.partial(
    pl.kernel,
    out_type=jax.ShapeDtypeStruct((2, NW, NP), jnp.float32),
    mesh=_mesh,
    compiler_params=_SC_PARAMS,
    scratch_types=[
        pltpu.VMEM((RD, K), jnp.int32),
        pltpu.VMEM((RD, K), jnp.int32),
        pltpu.VMEM((NP,), jnp.float32),
        pltpu.VMEM((NP,), jnp.float32),
    ],
)
def _sc_degrees(edges, out, srcv, dstv, dego, degi):
    cid = lax.axis_index("c")
    sid = lax.axis_index("s")
    wid = cid * NS + sid
    pltpu.sync_copy(edges.at[0, wid], srcv)
    pltpu.sync_copy(edges.at[1, wid], dstv)
    _zero_1d(dego, NP)
    _zero_1d(degi, NP)
    ones = jnp.ones((16,), jnp.float32)
    lanes = lax.iota(jnp.int32, 16)
    tile_base = wid * (RD * K)

    def body(m, _):
        for q in range(K // 16):
            pos = tile_base + m * K + q * 16 + lanes
            live = pos < E
            s16 = srcv[m, pl.ds(q * 16, 16)]
            d16 = dstv[m, pl.ds(q * 16, 16)]
            plsc.addupdate_scatter(dego, [s16], ones, mask=live)
            plsc.addupdate_scatter(degi, [d16], ones, mask=live)
        return 0

    lax.fori_loop(0, RD, body, 0)
    pltpu.sync_copy(dego, out.at[0, wid])
    pltpu.sync_copy(degi, out.at[1, wid])


def _norm_body(dq_ref, f_ref, isq_ref, h0n_ref):
    s = jnp.sum(dq_ref[...], axis=1)
    isq = lax.rsqrt(jnp.maximum(s, 1.0))
    isq_ref[...] = isq
    h0n_ref[...] = f_ref[...] * isq[0][:, None]


def _tc_norm(deg_p, features_p):
    bq = NP // 5
    return pl.pallas_call(
        _norm_body,
        grid=(5,),
        in_specs=[
            pl.BlockSpec((2, NW, bq), lambda i: (0, 0, i)),
            pl.BlockSpec((bq, D), lambda i: (i, 0)),
        ],
        out_specs=[
            pl.BlockSpec((2, bq), lambda i: (0, i)),
            pl.BlockSpec((bq, D), lambda i: (i, 0)),
        ],
        out_shape=[
            jax.ShapeDtypeStruct((2, NP), jnp.float32),
            jax.ShapeDtypeStruct((NP, D), jnp.float32),
        ],
    )(deg_p, features_p)


@functools.partial(
    pl.kernel,
    out_type=jax.ShapeDtypeStruct((NW, NP), jnp.float32),
    mesh=_mesh,
    compiler_params=_SC_PARAMS,
    scratch_types=[
        pltpu.VMEM((RD, K), jnp.int32),
        pltpu.VMEM((RD, K), jnp.int32),
        pltpu.VMEM((NP + 16,), jnp.float32),
        pltpu.VMEM((NP,), jnp.float32),
    ],
)
def _sc_cvec(edges, isq, c_out, srcv, dstv, iiv, cpart):
    cid = lax.axis_index("c")
    sid = lax.axis_index("s")
    wid = cid * NS + sid
    pltpu.sync_copy(edges.at[0, wid], srcv)
    pltpu.sync_copy(edges.at[1, wid], dstv)
    pltpu.sync_copy(isq.at[1], iiv.at[pl.ds(0, NP)])
    iiv[pl.ds(NP, 16)] = jnp.zeros((16,), jnp.float32)
    _zero_1d(cpart, NP)

    lc = jnp.clip(LCK - wid * RD, 0, RD)

    def body(m, _):
        for q in range(K // 16):
            s16 = srcv[m, pl.ds(q * 16, 16)]
            d16 = dstv[m, pl.ds(q * 16, 16)]
            wv = plsc.load_gather(iiv, [d16])
            plsc.addupdate_scatter(cpart, [s16], wv)
        return 0

    lax.fori_loop(0, lc, body, 0)
    pltpu.sync_copy(cpart, c_out.at[wid])


@functools.partial(
    pl.kernel,
    out_type=jax.ShapeDtypeStruct((NC, NP, D), jnp.float32),
    mesh=_mesh,
    compiler_params=_SC_PARAMS,
    scratch_types=[
        pltpu.VMEM((2 * SEGW, K), jnp.int32),
        pltpu.VMEM((2 * SEGW, K), jnp.int32),
        pltpu.VMEM((K, D), jnp.float32),
        pltpu.VMEM((K, D), jnp.float32),
        pltpu.VMEM_SHARED((ASH, D), jnp.float32),
        pltpu.SemaphoreType.DMA,
        pltpu.SemaphoreType.DMA,
    ],
)
def _sc_aggregate(edges, h0n, agg_out, sidx, didx, rows0, rows1, agg_sh,
                  gsem, ssem):
    cid = lax.axis_index("c")
    sid = lax.axis_index("s")
    wid = cid * NS + sid
    z16 = jnp.zeros((16,), jnp.float32)

    def zrow(r, _):
        for q in range(D // 16):
            rows0[r, pl.ds(q * 16, 16)] = z16
        return 0

    lax.fori_loop(0, K, zrow, 0)
    for blk in range(AZT // K):
        pltpu.sync_copy(rows0, agg_sh.at[pl.ds(sid * AZT + blk * K, K)])
    pltpu.sync_copy(rows0.at[pl.ds(0, AZT - (AZT // K) * K)],
                    agg_sh.at[pl.ds(sid * AZT + (AZT // K) * K,
                                    AZT - (AZT // K) * K)])
    plsc.subcore_barrier()

    lc = jnp.clip(LCK - wid * RD, 0, RD)
    rows = [rows0, rows1]

    def ldseg(s):
        h = jnp.bitwise_and(s, 1) * SEGW
        pltpu.sync_copy(edges.at[0, wid, pl.ds(s * SEGW, SEGW)],
                        sidx.at[pl.ds(h, SEGW)])
        pltpu.sync_copy(edges.at[1, wid, pl.ds(s * SEGW, SEGW)],
                        didx.at[pl.ds(h, SEGW)])
        return 0

    def gath(c, b):
        return pltpu.async_copy(
            h0n.at[sidx.at[jnp.bitwise_and(c, 2 * SEGW - 1)]], rows[b],
            gsem)

    def scat(c, b):
        return pltpu.async_copy(
            rows[b], agg_sh.at[didx.at[jnp.bitwise_and(c, 2 * SEGW - 1)]],
            ssem, add=True)

    def drain(sem):
        pltpu.make_async_copy(h0n.at[pl.ds(0, K)], rows0, sem).wait()
        return 0

    ldseg(0)

    def wave(w, _):
        c0 = 2 * w
        lax.cond(c0 >= 2, lambda: drain(ssem), lambda: 0)
        lax.cond(jnp.logical_and(jnp.bitwise_and(c0, SEGW - 1) == 0,
                                 c0 > 0),
                 lambda: ldseg(c0 // SEGW), lambda: 0)
        gath(c0, 0)
        lax.cond(c0 >= 1,
                 lambda: (drain(gsem), scat(c0 - 1, 1), 0)[2],
                 lambda: 0)
        lax.cond(c0 >= 1, lambda: drain(ssem), lambda: 0)
        gath(c0 + 1, 1)
        drain(gsem)
        scat(c0, 0)
        return 0

    lax.fori_loop(0, lc // 2, wave, 0)
    drain(gsem)
    scat_last = pltpu.async_copy(
        rows1, agg_sh.at[didx.at[jnp.bitwise_and(lc - 1, 2 * SEGW - 1)]],
        ssem, add=True)
    drain(ssem)
    scat_last.wait()
    plsc.subcore_barrier()
    pltpu.sync_copy(agg_sh.at[pl.ds(sid * (NP // NS), NP // NS)],
                    agg_out.at[cid, pl.ds(sid * (NP // NS), NP // NS)])


def _finish_body(agg_ref, ii_ref, oi_ref, c_ref, w1_ref, b1_ref, w2_ref,
                 b2_ref, out_ref, s_ref):
    i = pl.program_id(0)

    @pl.when(i == 0)
    def _():
        s_ref[...] = jnp.zeros_like(s_ref)

    aggn = (agg_ref[0] + agg_ref[1]) * ii_ref[...]
    h1 = jnp.maximum(
        jnp.dot(aggn, w1_ref[...], precision=_HIGH) + b1_ref[...], 0.0)
    h1n = h1 * oi_ref[...]
    cs = jnp.sum(c_ref[...], axis=0, keepdims=True)
    s_ref[...] += jnp.dot(cs, h1n, precision=_HIGH)

    @pl.when(i == pl.num_programs(0) - 1)
    def _():
        out_ref[...] = (
            jnp.dot(s_ref[...] * (1.0 / N), w2_ref[...], precision=_HIGH)
            + b2_ref[...])


def _tc_finish(agg, ii_col, oi_col, c_p, W1, b1r, W2, b2r):
    br = 512
    return pl.pallas_call(
        _finish_body,
        grid=(NP // br,),
        in_specs=[
            pl.BlockSpec((NC, br, D), lambda i: (0, i, 0)),
            pl.BlockSpec((br, 1), lambda i: (i, 0)),
            pl.BlockSpec((br, 1), lambda i: (i, 0)),
            pl.BlockSpec((NW, br), lambda i: (0, i)),
            pl.BlockSpec((D, D), lambda i: (0, 0)),
            pl.BlockSpec((1, D), lambda i: (0, 0)),
            pl.BlockSpec((D, D), lambda i: (0, 0)),
            pl.BlockSpec((1, D), lambda i: (0, 0)),
        ],
        out_specs=pl.BlockSpec((1, D), lambda i: (0, 0)),
        out_shape=jax.ShapeDtypeStruct((1, D), jnp.float32),
        scratch_shapes=[pltpu.VMEM((1, D), jnp.float32)],
    )(agg, ii_col, oi_col, c_p, W1, b1r, W2, b2r)


@jax.jit
def kernel(features, edge_index, W1, b1, W2, b2):
    ei = edge_index.astype(jnp.int32)
    ep = jnp.concatenate(
        [ei, jnp.stack([jnp.zeros((EP - E,), jnp.int32),
                        jnp.full((EP - E,), NP, jnp.int32)])], axis=1)
    ep = ep.reshape(2, NW, RD, K)
    deg_p = _sc_degrees(ep)
    features_p = jnp.pad(features, ((0, NP - N), (0, 0)))
    isq, h0n = _tc_norm(deg_p, features_p)
    c_p = _sc_cvec(ep, isq)
    agg = _sc_aggregate(ep, h0n)
    ii_col = isq[1].reshape(NP, 1)
    oi_col = isq[0].reshape(NP, 1)
    return _tc_finish(agg, ii_col, oi_col, c_p, W1,
                      b1.reshape(1, D), W2, b2.reshape(1, D))


# speedup vs baseline: 2.8832x; 1.0738x over previous
import functools

import jax
import jax.numpy as jnp
from jax import lax
from jax.experimental import pallas as pl
from jax.experimental.pallas import tpu as pltpu
from jax.experimental.pallas import tpu_sc as plsc

N = 10000
E = 320000
D = 128
NP = 10240
NC = 2
NS = 16
NW = NC * NS
EP = 327680
K = 128
RD = 80
LCK = E // K
AK = 80
ARD = 128
ASEG = 16
ASH = NP
AZT = ASH // NS

_mesh = plsc.VectorSubcoreMesh(core_axis_name="c", subcore_axis_name="s")
_HIGH = lax.Precision.HIGHEST
_SC_PARAMS = pltpu.CompilerParams(needs_layout_passes=False)


def _zero_1d(ref, n):
    z = jnp.zeros((16,), ref.dtype)

    def body(i, _):
        ref[pl.ds(i * 16, 16)] = z
        return 0

    lax.fori_loop(0, n // 16, body, 0)


@functools.partial(
    pl.kernel,
    out_type=jax.ShapeDtypeStruct((2, NW, NP), jnp.float32),
    mesh=_mesh,
    compiler_params=_SC_PARAMS,
    scratch_types=[
        pltpu.VMEM((RD, K), jnp.int32),
        pltpu.VMEM((RD, K), jnp.int32),
        pltpu.VMEM((NP,), jnp.float32),
        pltpu.VMEM((NP,), jnp.float32),
    ],
)
def _sc_degrees(edges, out, srcv, dstv, dego, degi):
    cid = lax.axis_index("c")
    sid = lax.axis_index("s")
    wid = cid * NS + sid
    pltpu.sync_copy(edges.at[0, wid], srcv)
    pltpu.sync_copy(edges.at[1, wid], dstv)
    _zero_1d(dego, NP)
    _zero_1d(degi, NP)
    ones = jnp.ones((16,), jnp.float32)
    lanes = lax.iota(jnp.int32, 16)
    tile_base = wid * (RD * K)

    def body(m, _):
        for q in range(K // 16):
            pos = tile_base + m * K + q * 16 + lanes
            live = pos < E
            s16 = srcv[m, pl.ds(q * 16, 16)]
            d16 = dstv[m, pl.ds(q * 16, 16)]
            plsc.addupdate_scatter(dego, [s16], ones, mask=live)
            plsc.addupdate_scatter(degi, [d16], ones, mask=live)
        return 0

    lax.fori_loop(0, RD, body, 0)
    pltpu.sync_copy(dego, out.at[0, wid])
    pltpu.sync_copy(degi, out.at[1, wid])


def _norm_body(dq_ref, f_ref, isq_ref, h0n_ref):
    s = jnp.sum(dq_ref[...], axis=1)
    isq = lax.rsqrt(jnp.maximum(s, 1.0))
    isq_ref[...] = isq
    h0n_ref[...] = f_ref[...] * isq[0][:, None]


def _tc_norm(deg_p, features_p):
    bq = NP // 5
    return pl.pallas_call(
        _norm_body,
        grid=(5,),
        in_specs=[
            pl.BlockSpec((2, NW, bq), lambda i: (0, 0, i)),
            pl.BlockSpec((bq, D), lambda i: (i, 0)),
        ],
        out_specs=[
            pl.BlockSpec((2, bq), lambda i: (0, i)),
            pl.BlockSpec((bq, D), lambda i: (i, 0)),
        ],
        out_shape=[
            jax.ShapeDtypeStruct((2, NP), jnp.float32),
            jax.ShapeDtypeStruct((NP, D), jnp.float32),
        ],
    )(deg_p, features_p)


@functools.partial(
    pl.kernel,
    out_type=jax.ShapeDtypeStruct((NW, NP), jnp.float32),
    mesh=_mesh,
    compiler_params=_SC_PARAMS,
    scratch_types=[
        pltpu.VMEM((RD, K), jnp.int32),
        pltpu.VMEM((RD, K), jnp.int32),
        pltpu.VMEM((NP + 16,), jnp.float32),
        pltpu.VMEM((NP,), jnp.float32),
    ],
)
def _sc_cvec(edges, isq, c_out, srcv, dstv, iiv, cpart):
    cid = lax.axis_index("c")
    sid = lax.axis_index("s")
    wid = cid * NS + sid
    pltpu.sync_copy(edges.at[0, wid], srcv)
    pltpu.sync_copy(edges.at[1, wid], dstv)
    pltpu.sync_copy(isq.at[1], iiv.at[pl.ds(0, NP)])
    iiv[pl.ds(NP, 16)] = jnp.zeros((16,), jnp.float32)
    _zero_1d(cpart, NP)

    lc = jnp.clip(LCK - wid * RD, 0, RD)

    def body(m, _):
        for q in range(K // 16):
            s16 = srcv[m, pl.ds(q * 16, 16)]
            d16 = dstv[m, pl.ds(q * 16, 16)]
            wv = plsc.load_gather(iiv, [d16])
            plsc.addupdate_scatter(cpart, [s16], wv)
        return 0

    lax.fori_loop(0, lc, body, 0)
    pltpu.sync_copy(cpart, c_out.at[wid])


@functools.partial(
    pl.kernel,
    out_type=jax.ShapeDtypeStruct((NC, NP, D), jnp.float32),
    mesh=_mesh,
    compiler_params=_SC_PARAMS,
    scratch_types=[
        pltpu.VMEM((2 * ASEG, AK), jnp.int32),
        pltpu.VMEM((2 * ASEG, AK), jnp.int32),
        pltpu.VMEM((AK, D), jnp.float32),
        pltpu.VMEM((AK, D), jnp.float32),
        pltpu.VMEM((AK, D), jnp.float32),
        pltpu.VMEM((AK, D), jnp.float32),
        pltpu.VMEM_SHARED((ASH, D), jnp.float32),
        pltpu.SemaphoreType.DMA,
        pltpu.SemaphoreType.DMA,
    ],
)
def _sc_aggregate(edges, h0n, agg_out, sidx, didx, rows0, rows1, rows2,
                  rows3, agg_sh, gsem, ssem):
    cid = lax.axis_index("c")
    sid = lax.axis_index("s")
    wid = cid * NS + sid
    z16 = jnp.zeros((16,), jnp.float32)

    def zrow(r, _):
        for q in range(D // 16):
            rows0[r, pl.ds(q * 16, 16)] = z16
            rows1[r, pl.ds(q * 16, 16)] = z16
        return 0

    lax.fori_loop(0, AK, zrow, 0)
    for blk in range(AZT // (2 * AK)):
        pltpu.sync_copy(rows0, agg_sh.at[pl.ds(sid * AZT + 2 * blk * AK,
                                               AK)])
        pltpu.sync_copy(rows1, agg_sh.at[pl.ds(
            sid * AZT + (2 * blk + 1) * AK, AK)])
    plsc.subcore_barrier()

    lc = jnp.clip(E // AK - wid * ARD, 0, ARD)
    rows = [rows0, rows1, rows2, rows3]

    def ldseg(s):
        h = jnp.bitwise_and(s, 1) * ASEG
        pltpu.sync_copy(edges.at[0, wid, pl.ds(s * ASEG, ASEG)],
                        sidx.at[pl.ds(h, ASEG)])
        pltpu.sync_copy(edges.at[1, wid, pl.ds(s * ASEG, ASEG)],
                        didx.at[pl.ds(h, ASEG)])
        return 0

    def gath(c, b):
        return pltpu.async_copy(
            h0n.at[sidx.at[jnp.bitwise_and(c, 2 * ASEG - 1)]], rows[b],
            gsem)

    def scat(c, b):
        return pltpu.async_copy(
            rows[b], agg_sh.at[didx.at[jnp.bitwise_and(c, 2 * ASEG - 1)]],
            ssem, add=True)

    def drain(sem):
        pltpu.make_async_copy(h0n.at[pl.ds(0, AK)], rows0, sem).wait()
        return 0

    ldseg(0)

    def it(t, _):
        c0 = 4 * t
        lax.cond(t > 0, lambda: drain(ssem), lambda: 0)
        lax.cond(jnp.logical_and(jnp.bitwise_and(c0, ASEG - 1) == 0,
                                 c0 > 0),
                 lambda: ldseg(c0 // ASEG), lambda: 0)
        gath(c0, 0)
        lax.cond(t > 0,
                 lambda: (drain(gsem), scat(c0 - 2, 2), 0)[2],
                 lambda: 0)
        lax.cond(t > 0, lambda: drain(ssem), lambda: 0)
        gath(c0 + 1, 1)
        lax.cond(t > 0,
                 lambda: (drain(gsem), scat(c0 - 1, 3), 0)[2],
                 lambda: 0)
        lax.cond(t > 0, lambda: drain(ssem), lambda: 0)
        gath(c0 + 2, 2)
        drain(gsem)
        scat(c0, 0)
        lax.cond(t > 0, lambda: drain(ssem), lambda: 0)
        gath(c0 + 3, 3)
        drain(gsem)
        scat(c0 + 1, 1)
        return 0

    lax.fori_loop(0, lc // 4, it, 0)
    drain(ssem)
    drain(gsem)
    scat(lc - 2, 2)
    drain(ssem)
    drain(gsem)
    scat(lc - 1, 3)
    drain(ssem)
    drain(ssem)
    plsc.subcore_barrier()
    pltpu.sync_copy(agg_sh.at[pl.ds(sid * (NP // NS), NP // NS)],
                    agg_out.at[cid, pl.ds(sid * (NP // NS), NP // NS)])


def _finish_body(agg_ref, ii_ref, oi_ref, c_ref, w1_ref, b1_ref, w2_ref,
                 b2_ref, out_ref, s_ref):
    i = pl.program_id(0)

    @pl.when(i == 0)
    def _():
        s_ref[...] = jnp.zeros_like(s_ref)

    aggn = (agg_ref[0] + agg_ref[1]) * ii_ref[...]
    h1 = jnp.maximum(
        jnp.dot(aggn, w1_ref[...], precision=_HIGH) + b1_ref[...], 0.0)
    h1n = h1 * oi_ref[...]
    cs = jnp.sum(c_ref[...], axis=0, keepdims=True)
    s_ref[...] += jnp.dot(cs, h1n, precision=_HIGH)

    @pl.when(i == pl.num_programs(0) - 1)
    def _():
        out_ref[...] = (
            jnp.dot(s_ref[...] * (1.0 / N), w2_ref[...], precision=_HIGH)
            + b2_ref[...])


def _tc_finish(agg, ii_col, oi_col, c_p, W1, b1r, W2, b2r):
    br = 512
    return pl.pallas_call(
        _finish_body,
        grid=(NP // br,),
        in_specs=[
            pl.BlockSpec((NC, br, D), lambda i: (0, i, 0)),
            pl.BlockSpec((br, 1), lambda i: (i, 0)),
            pl.BlockSpec((br, 1), lambda i: (i, 0)),
            pl.BlockSpec((NW, br), lambda i: (0, i)),
            pl.BlockSpec((D, D), lambda i: (0, 0)),
            pl.BlockSpec((1, D), lambda i: (0, 0)),
            pl.BlockSpec((D, D), lambda i: (0, 0)),
            pl.BlockSpec((1, D), lambda i: (0, 0)),
        ],
        out_specs=pl.BlockSpec((1, D), lambda i: (0, 0)),
        out_shape=jax.ShapeDtypeStruct((1, D), jnp.float32),
        scratch_shapes=[pltpu.VMEM((1, D), jnp.float32)],
    )(agg, ii_col, oi_col, c_p, W1, b1r, W2, b2r)


@jax.jit
def kernel(features, edge_index, W1, b1, W2, b2):
    ei = edge_index.astype(jnp.int32)
    ep = jnp.concatenate(
        [ei, jnp.stack([jnp.zeros((EP - E,), jnp.int32),
                        jnp.full((EP - E,), NP, jnp.int32)])], axis=1)
    epk = ep.reshape(2, NW, RD, K)
    epa = ep.reshape(2, NW, ARD, AK)
    deg_p = _sc_degrees(epk)
    features_p = jnp.pad(features, ((0, NP - N), (0, 0)))
    isq, h0n = _tc_norm(deg_p, features_p)
    c_p = _sc_cvec(epk, isq)
    agg = _sc_aggregate(epa, h0n)
    ii_col = isq[1].reshape(NP, 1)
    oi_col = isq[0].reshape(NP, 1)
    return _tc_finish(agg, ii_col, oi_col, c_p, W1,
                      b1.reshape(1, D), W2, b2.reshape(1, D))
